# baseline (speedup 1.0000x reference)
import functools

import jax
import jax.numpy as jnp
from jax import lax
from jax.experimental import pallas as pl
from jax.experimental.pallas import tpu as pltpu

N, E, B = 10000, 320000, 16
EBLK = 512
NBE = E // EBLK
NODE_BLK = 128
NBN = 79
NP = NBN * NODE_BLK
NPB = 80 * 128

NEG_INF = float("-inf")


def _mlp1_body(rel_ref, w1a_ref, b1a_ref, w1b_ref, b1b_ref, out_ref):
    relb = rel_ref[...]
    t = jnp.dot(relb, w1a_ref[...], preferred_element_type=jnp.float32)
    t = jnp.maximum(t + b1a_ref[...], 0.0)
    h = jnp.dot(t, w1b_ref[...], preferred_element_type=jnp.float32)
    out_ref[...] = h + b1b_ref[...]


def _mlp1(rel, w1a8, b1a, w1b, b1b):
    return pl.pallas_call(
        _mlp1_body,
        grid=(NBE,),
        in_specs=[
            pl.BlockSpec((EBLK, 8), lambda i: (i, 0)),
            pl.BlockSpec((8, 64), lambda i: (0, 0)),
            pl.BlockSpec((1, 64), lambda i: (0, 0)),
            pl.BlockSpec((64, 128), lambda i: (0, 0)),
            pl.BlockSpec((1, 128), lambda i: (0, 0)),
        ],
        out_specs=pl.BlockSpec((EBLK, 128), lambda i: (i, 0)),
        out_shape=jax.ShapeDtypeStruct((E, 128), jnp.float32),
    )(rel, w1a8, b1a, w1b, b1b)


def _tables_body(x_ref, pos_ref, w2ax_ref, w2ar_ref, b2a_ref, u_ref, p_ref):
    pb = jnp.dot(pos_ref[...], w2ar_ref[...], preferred_element_type=jnp.float32)
    ub = jnp.dot(x_ref[...], w2ax_ref[...], preferred_element_type=jnp.float32)
    p_ref[...] = pb
    u_ref[...] = ub + pb + b2a_ref[...]


def _tables(xp, posp, w2ax, w2ar8, b2a):
    return pl.pallas_call(
        _tables_body,
        grid=(NBN,),
        in_specs=[
            pl.BlockSpec((NODE_BLK, 128), lambda i: (i, 0)),
            pl.BlockSpec((NODE_BLK, 8), lambda i: (i, 0)),
            pl.BlockSpec((128, 256), lambda i: (0, 0)),
            pl.BlockSpec((8, 256), lambda i: (0, 0)),
            pl.BlockSpec((1, 256), lambda i: (0, 0)),
        ],
        out_specs=[
            pl.BlockSpec((NODE_BLK, 256), lambda i: (i, 0)),
            pl.BlockSpec((NODE_BLK, 256), lambda i: (i, 0)),
        ],
        out_shape=[
            jax.ShapeDtypeStruct((NP, 256), jnp.float32),
            jax.ShapeDtypeStruct((NP, 256), jnp.float32),
        ],
    )(xp, posp, w2ax, w2ar8, b2a)


def _mlp2_body(v_ref, dst_ref, batch_ref, iso_ref, w2b_ref, b2b_ref, g_ref):
    i = pl.program_id(0)

    @pl.when(i == 0)
    def _init():
        g_ref[...] = jnp.full((B, 512), NEG_INF, jnp.float32)

    h2 = jnp.dot(v_ref[...], w2b_ref[...], preferred_element_type=jnp.float32)
    h2 = h2 + b2b_ref[...]
    dst = dst_ref[0]
    batch = batch_ref[...]

    lbl = jnp.zeros_like(dst)
    for b in range(1, B):
        start_b = jnp.sum((batch < b).astype(jnp.int32))
        lbl = lbl + (dst >= start_b).astype(jnp.int32)

    acc = g_ref[...]
    rows = []
    for b in range(B):
        mb = jnp.max(jnp.where(lbl == b, h2, NEG_INF), axis=0, keepdims=True)
        rows.append(mb)
    g_ref[...] = jnp.maximum(acc, jnp.concatenate(rows, axis=0))

    @pl.when(i == NBE - 1)
    def _fin():
        iso = iso_ref[...]
        g = g_ref[...]
        floors = []
        for b in range(B):
            has_iso = jnp.sum(iso * (batch == b).astype(jnp.int32)) > 0
            floors.append(jnp.where(has_iso, 0.0, NEG_INF).reshape(1, 1))
        floor = jnp.concatenate(floors, axis=0)
        g = jnp.maximum(g, floor)
        g_ref[...] = jnp.where(jnp.isfinite(g), g, 0.0)


def _mlp2_pool(v, dst3, batchp, isop, w2b, b2b):
    return pl.pallas_call(
        _mlp2_body,
        grid=(NBE,),
        in_specs=[
            pl.BlockSpec((EBLK, 256), lambda i: (i, 0)),
            pl.BlockSpec((1, EBLK, 1), lambda i: (i, 0, 0)),
            pl.BlockSpec((80, 128), lambda i: (0, 0)),
            pl.BlockSpec((80, 128), lambda i: (0, 0)),
            pl.BlockSpec((256, 512), lambda i: (0, 0)),
            pl.BlockSpec((1, 512), lambda i: (0, 0)),
        ],
        out_specs=pl.BlockSpec((B, 512), lambda i: (0, 0)),
        out_shape=jax.ShapeDtypeStruct((B, 512), jnp.float32),
    )(v, dst3, batchp, isop, w2b, b2b)


def kernel(pos, edge_index, batch, W1a, b1a, W1b, b1b, W2a, b2a, W2b, b2b):
    src = edge_index[0]
    dst = edge_index[1]

    posp = jnp.zeros((NP, 8), jnp.float32).at[:N, :3].set(pos)
    w1a8 = jnp.zeros((8, 64), jnp.float32).at[:3].set(W1a)
    w2ar8 = jnp.zeros((8, 256), jnp.float32).at[:3].set(W2a[128:131])

    rel = posp[src] - posp[dst]

    h = _mlp1(rel, w1a8, b1a.reshape(1, 64), W1b, b1b.reshape(1, 128))

    x = jax.ops.segment_max(h, dst, num_segments=N)
    iso = (jax.ops.segment_sum(jnp.ones((E,), jnp.int32), dst, num_segments=N)
           == 0)
    x = jnp.where(iso[:, None], 0.0, x)
    xp = jnp.zeros((NP, 128), jnp.float32).at[:N].set(x)

    u, p = _tables(xp, posp, W2a[:128], w2ar8, b2a.reshape(1, 256))

    v = jnp.maximum(u[src] - p[dst], 0.0)

    batchp = jnp.full((NPB,), B, jnp.int32).at[:N].set(batch).reshape(80, 128)
    isop = jnp.zeros((NPB,), jnp.int32).at[:N].set(iso.astype(jnp.int32))
    isop = isop.reshape(80, 128)
    dst3 = dst.reshape(NBE, EBLK, 1)

    g = _mlp2_pool(v, dst3, batchp, isop, W2b, b2b.reshape(1, 512))
    return g
